# Initial kernel scaffold; baseline (speedup 1.0000x reference)
import jax
import jax.numpy as jnp
from jax.experimental import pallas as pl


def kernel(s, z, edge_idx, valid_mask, res_type_clone, params):
    raise NotImplementedError("write your pallas kernel here")


# speedup vs baseline: 5.1487x; 5.1487x over previous
import functools

import jax
import jax.numpy as jnp
from jax.experimental import pallas as pl
from jax.experimental.pallas import tpu as pltpu

_INTERPRET = False

EB = 512
NB = 1000


def _gelu(x):
    return x * 0.5 * (1.0 + jax.lax.erf(x / jnp.sqrt(2.0).astype(x.dtype)))


def _node_pre_body(s0_ref, rtc_ref, rand_ref, seqWT_ref, wd0T_ref, b1a0_ref,
                   tab_ref, p0_ref):
    s0 = s0_ref[...]
    r = jnp.dot(rtc_ref[...], seqWT_ref[...],
                preferred_element_type=jnp.float32)
    tab_ref[:, 0:128] = s0
    tab_ref[:, 128:256] = r
    p0_ref[...] = jnp.dot(s0, wd0T_ref[...],
                          preferred_element_type=jnp.float32) + b1a0_ref[...]
    del rand_ref


def _node_pre(s0, rtc, rand, seqWT, wd0T, b1a0):
    n = s0.shape[0]
    grid = (n // NB,)
    tab, p0 = pl.pallas_call(
        _node_pre_body,
        grid=grid,
        in_specs=[
            pl.BlockSpec((NB, 128), lambda i: (i, 0)),
            pl.BlockSpec((NB, 33), lambda i: (i, 0)),
            pl.BlockSpec((NB, 1), lambda i: (i, 0)),
            pl.BlockSpec((33, 128), lambda i: (0, 0)),
            pl.BlockSpec((128, 128), lambda i: (0, 0)),
            pl.BlockSpec((1, 128), lambda i: (0, 0)),
        ],
        out_specs=[
            pl.BlockSpec((NB, 256), lambda i: (i, 0)),
            pl.BlockSpec((NB, 128), lambda i: (i, 0)),
        ],
        out_shape=[
            jax.ShapeDtypeStruct((n, 256), jnp.float32),
            jax.ShapeDtypeStruct((n, 128), jnp.float32),
        ],
        interpret=_INTERPRET,
    )(s0, rtc, rand[:, None], seqWT, wd0T, b1a0)
    return tab, p0


def _edge_body(first, z_ref, srg_ref, vis_ref, pd_ref,
               waT_ref, w2aT_ref, b2a_ref, w3aT_ref, b3a_ref,
               wvT_ref, b1v_ref, w2vT_ref, b2v_ref, w3vT_ref, b3v_ref,
               seqb_ref,
               expaw_ref, av_ref, u_ref):
    z = z_ref[...]
    if first:
        u = (srg_ref[:, 0:128]
             + vis_ref[...] * srg_ref[:, 128:256]
             + seqb_ref[...])
        u_ref[...] = u
    else:
        u = srg_ref[...]
    zu = jnp.concatenate([z, u], axis=1)
    h = jnp.dot(zu, waT_ref[...], preferred_element_type=jnp.float32) + pd_ref[...]
    h = _gelu(h)
    h = _gelu(jnp.dot(h, w2aT_ref[...], preferred_element_type=jnp.float32)
              + b2a_ref[...])
    aw = jnp.dot(h, w3aT_ref[...], preferred_element_type=jnp.float32) + b3a_ref[...]
    expaw_ref[...] = jnp.exp(aw)
    g = _gelu(jnp.dot(zu, wvT_ref[...], preferred_element_type=jnp.float32)
              + b1v_ref[...])
    g = _gelu(jnp.dot(g, w2vT_ref[...], preferred_element_type=jnp.float32)
              + b2v_ref[...])
    av_ref[...] = (jnp.dot(g, w3vT_ref[...], preferred_element_type=jnp.float32)
                   + b3v_ref[...])


def _edge_kernel(first, z, srg, vis, pd, lw, seqb):
    e = z.shape[0]
    grid = (e // EB,)
    (waT, w2aT, b2a, w3aT, b3a, wvT, b1v, w2vT, b2v, w3vT, b3v) = lw
    full = lambda shape: pl.BlockSpec(shape, lambda i: tuple(0 for _ in shape))
    srg_cols = srg.shape[1]
    outs = pl.pallas_call(
        functools.partial(_edge_body, first),
        grid=grid,
        in_specs=[
            pl.BlockSpec((EB, 128), lambda i: (i, 0)),
            pl.BlockSpec((EB, srg_cols), lambda i: (i, 0)),
            pl.BlockSpec((EB, 1), lambda i: (i, 0)),
            pl.BlockSpec((EB, 128), lambda i: (i, 0)),
            full((256, 128)), full((128, 128)), full((1, 128)),
            full((128, 4)), full((1, 4)),
            full((256, 128)), full((1, 128)), full((128, 128)), full((1, 128)),
            full((128, 128)), full((1, 128)),
            full((1, 128)),
        ],
        out_specs=[
            pl.BlockSpec((EB, 4), lambda i: (i, 0)),
            pl.BlockSpec((EB, 128), lambda i: (i, 0)),
            pl.BlockSpec((EB, 128), lambda i: (i, 0)),
        ],
        out_shape=[
            jax.ShapeDtypeStruct((e, 4), jnp.float32),
            jax.ShapeDtypeStruct((e, 128), jnp.float32),
            jax.ShapeDtypeStruct((e, 128), jnp.float32),
        ],
        interpret=_INTERPRET,
    )(z, srg, vis, pd, waT, w2aT, b2a, w3aT, b3a, wvT, b1v, w2vT, b2v,
      w3vT, b3v, seqb)
    return outs


def _update_body(last, num_ref, s_ref, woutT_ref, bout_ref,
                 wf1T_ref, bf1_ref, wf2T_ref, bf2_ref, wnT_ref, bn_ref,
                 s_out_ref, p_out_ref):
    inv = 1.0 / jnp.sqrt(1.0 + 1e-5)
    ao = jnp.concatenate(
        [(num_ref[0, h, :, 0:128] + num_ref[1, h, :, 0:128])
         / (num_ref[0, h, :, 128:129] + num_ref[1, h, :, 128:129] + 1e-12)
         for h in range(4)], axis=1)
    s = s_ref[...]
    s = s + (jnp.dot(ao, woutT_ref[...], preferred_element_type=jnp.float32)
             + bout_ref[...]) * inv
    t = _gelu(jnp.dot(s, wf1T_ref[...], preferred_element_type=jnp.float32)
              + bf1_ref[...])
    s = s + (jnp.dot(t, wf2T_ref[...], preferred_element_type=jnp.float32)
             + bf2_ref[...]) * inv
    s_out_ref[...] = s
    p = jnp.dot(s, wnT_ref[...], preferred_element_type=jnp.float32) + bn_ref[...]
    p_out_ref[...] = p
    del last


def _update_kernel(num, s, woutT, bout, wf1T, bf1, wf2T, bf2, wnT, bn):
    n = s.shape[0]
    pc = wnT.shape[1]
    grid = (n // NB,)
    full = lambda shape: pl.BlockSpec(shape, lambda i: tuple(0 for _ in shape))
    s_new, p_new = pl.pallas_call(
        functools.partial(_update_body, False),
        grid=grid,
        in_specs=[
            pl.BlockSpec((2, 4, NB, 144), lambda i: (0, 0, i, 0)),
            pl.BlockSpec((NB, 128), lambda i: (i, 0)),
            full((512, 128)), full((1, 128)),
            full((128, 128)), full((1, 128)),
            full((128, 128)), full((1, 128)),
            full((128, pc)), full((1, pc)),
        ],
        out_specs=[
            pl.BlockSpec((NB, 128), lambda i: (i, 0)),
            pl.BlockSpec((NB, pc), lambda i: (i, 0)),
        ],
        out_shape=[
            jax.ShapeDtypeStruct((n, 128), jnp.float32),
            jax.ShapeDtypeStruct((n, pc), jnp.float32),
        ],
        interpret=_INTERPRET,
    )(num, s, woutT, bout, wf1T, bf1, wf2T, bf2, wnT, bn)
    return s_new, p_new


def _gather_rows(tab, idx):
    return tab[idx]


def _vis_compute(rand, src, dst):
    return (rand[src] < rand[dst]).astype(jnp.float32)


def _aggregate(expaw, av, dst, n):
    e = expaw.shape[0]
    half = e // 2
    out = []
    for p in range(2):
        sl = slice(p * half, (p + 1) * half)
        vals = jnp.concatenate(
            [av[sl], jnp.ones((half, 1), jnp.float32),
             jnp.zeros((half, 15), jnp.float32)], axis=1)
        per_h = []
        for h in range(4):
            per_h.append(jax.ops.segment_sum(
                expaw[sl, h:h + 1] * vals, dst[sl], num_segments=n))
        out.append(jnp.stack(per_h, axis=0))
    return jnp.stack(out, axis=0)


def kernel(s, z, edge_idx, valid_mask, res_type_clone, params):
    n, d = s.shape
    kk = res_type_clone.shape[-1]
    src, dst = edge_idx[0], edge_idx[1]
    rand = jax.random.uniform(jax.random.key(42), (n,), dtype=s.dtype)
    rtc = (res_type_clone != 0).reshape(-1, kk).astype(s.dtype)

    seqW, seqb = params["seq_to_s"]
    layers = params["layers"]

    def lt(p):
        W, b = p
        return W.T, b[None, :]

    lws = []
    for lp in layers:
        w1a, b1a = lp["aw"][0]
        w2aT, b2a = lt(lp["aw"][1])
        w3aT, b3a = lt(lp["aw"][2])
        wvT, b1v = lt(lp["av"][0])
        w2vT, b2v = lt(lp["av"][1])
        w3vT, b3v = lt(lp["av"][2])
        wdT = w1a[:, 0:128].T
        waT = w1a[:, 128:384].T
        lws.append(dict(wdT=wdT, b1a=b1a[None, :],
                        ew=(waT, w2aT, b2a, w3aT, b3a,
                            wvT, b1v, w2vT, b2v, w3vT, b3v)))

    tab, p = _node_pre(s, rtc, rand, seqW.T, lws[0]["wdT"], lws[0]["b1a"])

    vis = _vis_compute(rand, src, dst)[:, None]
    srg0 = _gather_rows(tab, src)

    u = None
    cur_s = s
    for li, lp in enumerate(layers):
        pd = _gather_rows(p, dst)
        if li == 0:
            expaw, av, u = _edge_kernel(True, z, srg0, vis, pd,
                                        lws[li]["ew"], seqb[None, :])
        else:
            expaw, av, _ = _edge_kernel(False, z, u, vis, pd,
                                        lws[li]["ew"], seqb[None, :])
        num = _aggregate(expaw, av, dst, n)
        woutT, bout = lt(layers[li]["out"])
        wf1T, bf1 = lt(layers[li]["ffn"][0])
        wf2T, bf2 = lt(layers[li]["ffn"][1])
        if li + 1 < len(layers):
            wnT, bn = lws[li + 1]["wdT"], lws[li + 1]["b1a"]
        else:
            predW = params["pred_W"]
            wnT, bn = predW.T, jnp.zeros((1, predW.shape[0]), jnp.float32)
        cur_s, p = _update_kernel(num, cur_s, woutT, bout, wf1T, bf1,
                                  wf2T, bf2, wnT, bn)

    logits = p
    bm, nn = valid_mask.shape
    return logits.reshape(bm, nn, kk)


# speedup vs baseline: 6.7118x; 1.3036x over previous
import functools

import jax
import jax.numpy as jnp
from jax import lax
from jax.experimental import pallas as pl
from jax.experimental.pallas import tpu as pltpu
from jax.experimental.pallas import tpu_sc as plsc

_INTERPRET = False

EB = 512
NB = 1000
NW = 32
CH = 80


def _sc_mesh():
    return plsc.VectorSubcoreMesh(core_axis_name="c", subcore_axis_name="s")


def _sc_gather(tab, idx):
    e = idx.shape[0]
    cols = tab.shape[1]
    per_w = e // NW
    nch = per_w // CH

    @functools.partial(
        pl.kernel, mesh=_sc_mesh(),
        out_type=jax.ShapeDtypeStruct((e, cols), jnp.float32),
        scratch_types=[
            pltpu.VMEM((per_w,), jnp.int32),
            pltpu.VMEM((CH, cols), jnp.float32),
            pltpu.SemaphoreType.DMA,
        ],
    )
    def k(tab_hbm, idx_hbm, out_hbm, idx_v, rows_v, sem):
        wid = lax.axis_index("c") * 16 + lax.axis_index("s")
        base = wid * per_w
        pltpu.sync_copy(idx_hbm.at[pl.ds(base, per_w)], idx_v)

        def body(j, _):
            off = pl.multiple_of(j * CH, CH)
            pltpu.async_copy(tab_hbm.at[idx_v.at[pl.ds(off, CH)]],
                             rows_v, sem).wait()
            pltpu.sync_copy(rows_v, out_hbm.at[pl.ds(base + off, CH)])
            return 0

        lax.fori_loop(0, nch, body, 0)

    return k(tab, idx)


def _gelu(x):
    return x * 0.5 * (1.0 + jax.lax.erf(x / jnp.sqrt(2.0).astype(x.dtype)))


def _node_pre_body(s0_ref, rtc_ref, rand_ref, seqWT_ref, wd0T_ref, b1a0_ref,
                   tab_ref, p0_ref):
    s0 = s0_ref[...]
    rand = rand_ref[...]
    r = jnp.dot(rtc_ref[...], seqWT_ref[...],
                preferred_element_type=jnp.float32)
    tab_ref[:, 0:128] = s0
    tab_ref[:, 128:256] = r
    tab_ref[:, 256:257] = rand
    tab_ref[:, 257:384] = jnp.zeros_like(tab_ref[:, 257:384])
    p0_ref[:, 0:128] = jnp.dot(s0, wd0T_ref[...],
                               preferred_element_type=jnp.float32) + b1a0_ref[...]
    p0_ref[:, 128:129] = rand
    p0_ref[:, 129:256] = jnp.zeros_like(p0_ref[:, 129:256])


def _node_pre(s0, rtc, rand, seqWT, wd0T, b1a0):
    n = s0.shape[0]
    grid = (n // NB,)
    tab, p0 = pl.pallas_call(
        _node_pre_body,
        grid=grid,
        in_specs=[
            pl.BlockSpec((NB, 128), lambda i: (i, 0)),
            pl.BlockSpec((NB, 33), lambda i: (i, 0)),
            pl.BlockSpec((NB, 1), lambda i: (i, 0)),
            pl.BlockSpec((33, 128), lambda i: (0, 0)),
            pl.BlockSpec((128, 128), lambda i: (0, 0)),
            pl.BlockSpec((1, 128), lambda i: (0, 0)),
        ],
        out_specs=[
            pl.BlockSpec((NB, 384), lambda i: (i, 0)),
            pl.BlockSpec((NB, 256), lambda i: (i, 0)),
        ],
        out_shape=[
            jax.ShapeDtypeStruct((n, 384), jnp.float32),
            jax.ShapeDtypeStruct((n, 256), jnp.float32),
        ],
        interpret=_INTERPRET,
    )(s0, rtc, rand[:, None], seqWT, wd0T, b1a0)
    return tab, p0


def _edge_body(first, z_ref, srg_ref, pd_ref,
               waT_ref, w2aT_ref, b2a_ref, w3aT_ref, b3a_ref,
               wvT_ref, b1v_ref, w2vT_ref, b2v_ref, w3vT_ref, b3v_ref,
               seqb_ref,
               expaw_ref, av_ref, u_ref):
    z = z_ref[...]
    if first:
        vis = jnp.where(srg_ref[:, 256:257] < pd_ref[:, 128:129], 1.0, 0.0)
        u = (srg_ref[:, 0:128]
             + vis * srg_ref[:, 128:256]
             + seqb_ref[...])
        u_ref[...] = u
    else:
        u = srg_ref[...]
    zu = jnp.concatenate([z, u], axis=1)
    h = (jnp.dot(zu, waT_ref[...], preferred_element_type=jnp.float32)
         + pd_ref[:, 0:128])
    h = _gelu(h)
    h = _gelu(jnp.dot(h, w2aT_ref[...], preferred_element_type=jnp.float32)
              + b2a_ref[...])
    aw = jnp.dot(h, w3aT_ref[...], preferred_element_type=jnp.float32) + b3a_ref[...]
    expaw_ref[...] = jnp.exp(aw)
    g = _gelu(jnp.dot(zu, wvT_ref[...], preferred_element_type=jnp.float32)
              + b1v_ref[...])
    g = _gelu(jnp.dot(g, w2vT_ref[...], preferred_element_type=jnp.float32)
              + b2v_ref[...])
    av_ref[...] = (jnp.dot(g, w3vT_ref[...], preferred_element_type=jnp.float32)
                   + b3v_ref[...])


def _edge_kernel(first, z, srg, pd, lw, seqb):
    e = z.shape[0]
    grid = (e // EB,)
    (waT, w2aT, b2a, w3aT, b3a, wvT, b1v, w2vT, b2v, w3vT, b3v) = lw
    full = lambda shape: pl.BlockSpec(shape, lambda i: tuple(0 for _ in shape))
    srg_cols = srg.shape[1]
    pd_cols = pd.shape[1]
    outs = pl.pallas_call(
        functools.partial(_edge_body, first),
        grid=grid,
        in_specs=[
            pl.BlockSpec((EB, 128), lambda i: (i, 0)),
            pl.BlockSpec((EB, srg_cols), lambda i: (i, 0)),
            pl.BlockSpec((EB, pd_cols), lambda i: (i, 0)),
            full((256, 128)), full((128, 128)), full((1, 128)),
            full((128, 4)), full((1, 4)),
            full((256, 128)), full((1, 128)), full((128, 128)), full((1, 128)),
            full((128, 128)), full((1, 128)),
            full((1, 128)),
        ],
        out_specs=[
            pl.BlockSpec((EB, 4), lambda i: (i, 0)),
            pl.BlockSpec((EB, 128), lambda i: (i, 0)),
            pl.BlockSpec((EB, 128), lambda i: (i, 0)),
        ],
        out_shape=[
            jax.ShapeDtypeStruct((e, 4), jnp.float32),
            jax.ShapeDtypeStruct((e, 128), jnp.float32),
            jax.ShapeDtypeStruct((e, 128), jnp.float32),
        ],
        interpret=_INTERPRET,
    )(z, srg, pd, waT, w2aT, b2a, w3aT, b3a, wvT, b1v, w2vT, b2v,
      w3vT, b3v, seqb)
    return outs


def _update_body(last, num_ref, s_ref, woutT_ref, bout_ref,
                 wf1T_ref, bf1_ref, wf2T_ref, bf2_ref, wnT_ref, bn_ref,
                 s_out_ref, p_out_ref):
    inv = 1.0 / jnp.sqrt(1.0 + 1e-5)
    ao = jnp.concatenate(
        [(num_ref[0, h, :, 0:128] + num_ref[1, h, :, 0:128])
         / (num_ref[0, h, :, 128:129] + num_ref[1, h, :, 128:129] + 1e-12)
         for h in range(4)], axis=1)
    s = s_ref[...]
    s = s + (jnp.dot(ao, woutT_ref[...], preferred_element_type=jnp.float32)
             + bout_ref[...]) * inv
    t = _gelu(jnp.dot(s, wf1T_ref[...], preferred_element_type=jnp.float32)
              + bf1_ref[...])
    s = s + (jnp.dot(t, wf2T_ref[...], preferred_element_type=jnp.float32)
             + bf2_ref[...]) * inv
    s_out_ref[...] = s
    p = jnp.dot(s, wnT_ref[...], preferred_element_type=jnp.float32) + bn_ref[...]
    p_out_ref[...] = p
    del last


def _update_kernel(num, s, woutT, bout, wf1T, bf1, wf2T, bf2, wnT, bn):
    n = s.shape[0]
    pc = wnT.shape[1]
    grid = (n // NB,)
    full = lambda shape: pl.BlockSpec(shape, lambda i: tuple(0 for _ in shape))
    s_new, p_new = pl.pallas_call(
        functools.partial(_update_body, False),
        grid=grid,
        in_specs=[
            pl.BlockSpec((2, 4, NB, 144), lambda i: (0, 0, i, 0)),
            pl.BlockSpec((NB, 128), lambda i: (i, 0)),
            full((512, 128)), full((1, 128)),
            full((128, 128)), full((1, 128)),
            full((128, 128)), full((1, 128)),
            full((128, pc)), full((1, pc)),
        ],
        out_specs=[
            pl.BlockSpec((NB, 128), lambda i: (i, 0)),
            pl.BlockSpec((NB, pc), lambda i: (i, 0)),
        ],
        out_shape=[
            jax.ShapeDtypeStruct((n, 128), jnp.float32),
            jax.ShapeDtypeStruct((n, pc), jnp.float32),
        ],
        interpret=_INTERPRET,
    )(num, s, woutT, bout, wf1T, bf1, wf2T, bf2, wnT, bn)
    return s_new, p_new


def _gather_rows(tab, idx):
    return tab[idx]


def _vis_compute(rand, src, dst):
    return (rand[src] < rand[dst]).astype(jnp.float32)


def _aggregate(expaw, av, dst, n):
    e = expaw.shape[0]
    half = e // 2
    out = []
    for p in range(2):
        sl = slice(p * half, (p + 1) * half)
        vals = jnp.concatenate(
            [av[sl], jnp.ones((half, 1), jnp.float32),
             jnp.zeros((half, 15), jnp.float32)], axis=1)
        per_h = []
        for h in range(4):
            per_h.append(jax.ops.segment_sum(
                expaw[sl, h:h + 1] * vals, dst[sl], num_segments=n))
        out.append(jnp.stack(per_h, axis=0))
    return jnp.stack(out, axis=0)


def kernel(s, z, edge_idx, valid_mask, res_type_clone, params):
    n, d = s.shape
    kk = res_type_clone.shape[-1]
    src = edge_idx[0].astype(jnp.int32)
    dst = edge_idx[1].astype(jnp.int32)
    rand = jax.random.uniform(jax.random.key(42), (n,), dtype=s.dtype)
    rtc = (res_type_clone != 0).reshape(-1, kk).astype(s.dtype)

    seqW, seqb = params["seq_to_s"]
    layers = params["layers"]

    def lt(p):
        W, b = p
        return W.T, b[None, :]

    lws = []
    for lp in layers:
        w1a, b1a = lp["aw"][0]
        w2aT, b2a = lt(lp["aw"][1])
        w3aT, b3a = lt(lp["aw"][2])
        wvT, b1v = lt(lp["av"][0])
        w2vT, b2v = lt(lp["av"][1])
        w3vT, b3v = lt(lp["av"][2])
        wdT = w1a[:, 0:128].T
        waT = w1a[:, 128:384].T
        lws.append(dict(wdT=wdT, b1a=b1a[None, :],
                        ew=(waT, w2aT, b2a, w3aT, b3a,
                            wvT, b1v, w2vT, b2v, w3vT, b3v)))

    tab, p = _node_pre(s, rtc, rand, seqW.T, lws[0]["wdT"], lws[0]["b1a"])

    srg0 = _sc_gather(tab, src)

    u = None
    cur_s = s
    for li, lp in enumerate(layers):
        pd = _sc_gather(p, dst)
        if li == 0:
            expaw, av, u = _edge_kernel(True, z, srg0, pd,
                                        lws[li]["ew"], seqb[None, :])
        else:
            expaw, av, _ = _edge_kernel(False, z, u, pd,
                                        lws[li]["ew"], seqb[None, :])
        num = _aggregate(expaw, av, dst, n)
        woutT, bout = lt(layers[li]["out"])
        wf1T, bf1 = lt(layers[li]["ffn"][0])
        wf2T, bf2 = lt(layers[li]["ffn"][1])
        if li + 1 < len(layers):
            wnT, bn = lws[li + 1]["wdT"], lws[li + 1]["b1a"]
        else:
            predW = params["pred_W"]
            wnT, bn = predW.T, jnp.zeros((1, predW.shape[0]), jnp.float32)
        cur_s, p = _update_kernel(num, cur_s, woutT, bout, wf1T, bf1,
                                  wf2T, bf2, wnT, bn)

    logits = p
    bm, nn = valid_mask.shape
    return logits.reshape(bm, nn, kk)
